# baseline (speedup 1.0000x reference)
import functools

import jax
import jax.numpy as jnp
from jax import lax
from jax.experimental import pallas as pl
from jax.experimental.pallas import tpu as pltpu
from jax.experimental.pallas import tpu_sc as plsc

NUM_CORES = 2
NUM_SUBCORES = 16
LANES = 16
NUM_WORKERS = NUM_CORES * NUM_SUBCORES

BATCH = 16384
FACTORS = 32
B_PER_W = BATCH // NUM_WORKERS


def _make_sc_kernel():
  mesh = plsc.VectorSubcoreMesh(core_axis_name="c", subcore_axis_name="s")

  @functools.partial(
      pl.kernel,
      out_type=jax.ShapeDtypeStruct((BATCH,), jnp.float32),
      mesh=mesh,
      compiler_params=pltpu.CompilerParams(
          needs_layout_passes=False, use_tc_tiling_on_sc=False),
      scratch_types=[
          pltpu.VMEM((B_PER_W,), jnp.int32),
          pltpu.VMEM((B_PER_W,), jnp.int32),
          pltpu.VMEM((B_PER_W, FACTORS), jnp.float32),
          pltpu.VMEM((B_PER_W, FACTORS), jnp.float32),
          pltpu.VMEM((B_PER_W,), jnp.float32),
          pltpu.SemaphoreType.DMA,
          pltpu.SemaphoreType.DMA,
      ],
  )
  def dot_kernel(inv_idx_hbm, td_idx_hbm, inv_tab_hbm, td_tab_hbm, out_hbm,
                 idx_a, idx_b, rows_a, rows_b, out_v, sem_a, sem_b):
    wid = lax.axis_index("s") * NUM_CORES + lax.axis_index("c")
    base = wid * B_PER_W

    pltpu.sync_copy(inv_idx_hbm.at[pl.ds(base, B_PER_W)], idx_a)
    pltpu.sync_copy(td_idx_hbm.at[pl.ds(base, B_PER_W)], idx_b)

    cp_a = pltpu.async_copy(inv_tab_hbm.at[idx_a], rows_a, sem_a)
    cp_b = pltpu.async_copy(td_tab_hbm.at[idx_b], rows_b, sem_b)
    cp_a.wait()
    cp_b.wait()

    lane = lax.iota(jnp.int32, LANES)

    def group_body(g):
      row_ids = g * LANES + lane
      acc = jnp.zeros((LANES,), jnp.float32)
      for d in range(FACTORS):
        col = jnp.full((LANES,), d, jnp.int32)
        va = plsc.load_gather(rows_a, [row_ids, col])
        vb = plsc.load_gather(rows_b, [row_ids, col])
        acc = acc + va * vb
      out_v[pl.ds(g * LANES, LANES)] = acc

    pl.loop(0, B_PER_W // LANES)(group_body)

    pltpu.sync_copy(out_v, out_hbm.at[pl.ds(base, B_PER_W)])

  return dot_kernel


_sc_dot = _make_sc_kernel()


@jax.jit
def kernel(investor, ticker, date, ticker_date, investor_factors,
           ticker_date_factors):
  del ticker, date
  inv_idx = investor.astype(jnp.int32)
  td_idx = ticker_date.astype(jnp.int32)
  return _sc_dot(inv_idx, td_idx, investor_factors, ticker_date_factors)
